# baseline (speedup 1.0000x reference)
import jax
import jax.numpy as jnp
from jax.experimental import pallas as pl
from jax.experimental.pallas import tpu as pltpu


def _gx_t_kernel(x_ref, g_ref, o_ref):
    k = pl.program_id(1)

    @pl.when(k == 0)
    def _():
        o_ref[...] = jnp.zeros_like(o_ref)

    g = g_ref[...].astype(jnp.bfloat16)
    o_ref[...] += jax.lax.dot_general(
        x_ref[...], g,
        dimension_numbers=(((0,), (1,)), ((), ())),
        preferred_element_type=jnp.float32)


def _gx_t(g, x16, bm, bk):
    m, kdim = g.shape
    return pl.pallas_call(
        _gx_t_kernel,
        grid=(m // bm, kdim // bk),
        in_specs=[
            pl.BlockSpec((bk, 32), lambda i, k: (k, 0)),
            pl.BlockSpec((bm, bk), lambda i, k: (i, k)),
        ],
        out_specs=pl.BlockSpec((32, bm), lambda i, k: (0, i)),
        out_shape=jax.ShapeDtypeStruct((32, m), jnp.float32),
        compiler_params=pltpu.CompilerParams(
            dimension_semantics=("arbitrary", "arbitrary"),
        ),
    )(x16, g)


def _w_apply_kernel(tv_ref, te1_ref, te2_ref, tf1_ref, tf2_ref,
                    wv_ref, we1_ref, we2_ref, wf1_ref, wf2_ref,
                    ov_ref, oe_ref, of_ref):
    def wt(w_ref, t_ref):
        return jax.lax.dot_general(
            w_ref[...].astype(jnp.bfloat16),
            t_ref[...].astype(jnp.bfloat16),
            dimension_numbers=(((0,), (0,)), ((), ())),
            preferred_element_type=jnp.float32)

    ov_ref[...] = wt(wv_ref, tv_ref)
    oe_ref[...] = wt(we1_ref, te1_ref) + wt(we2_ref, te2_ref)
    of_ref[...] = wt(wf1_ref, tf1_ref) + wt(wf2_ref, tf2_ref)


def _w_apply(tv, te1, te2, tf1, tf2, wv, we1, we2, wf1, wf2):
    nv = tv.shape[1]
    ne = te1.shape[1]
    nf = tf1.shape[1]
    return pl.pallas_call(
        _w_apply_kernel,
        out_shape=(
            jax.ShapeDtypeStruct((32, nv), jnp.float32),
            jax.ShapeDtypeStruct((32, ne), jnp.float32),
            jax.ShapeDtypeStruct((32, nf), jnp.float32),
        ),
    )(tv, te1, te2, tf1, tf2, wv, we1, we2, wf1, wf2)


@jax.jit
def kernel(xv, xe, xf, Gv2v, Gv2e, Ge2e, Ge2f, Gf2f, Wv2v, Wve, Wee, Wef, Wff):
    xv16 = xv.astype(jnp.bfloat16)
    xe16 = xe.astype(jnp.bfloat16)
    xf16 = xf.astype(jnp.bfloat16)

    bm, bk = 1024, 4096
    tv = _gx_t(Gv2v, xv16, bm, bk)
    te1 = _gx_t(Gv2e, xv16, bm, bk)
    te2 = _gx_t(Ge2e, xe16, bm, bk)
    tf1 = _gx_t(Ge2f, xe16, bm, bk)
    tf2 = _gx_t(Gf2f, xf16, bm, bk)

    zvt, zet, zft = _w_apply(tv, te1, te2, tf1, tf2,
                             Wv2v, Wve, Wee, Wef, Wff)
    return (zvt.T, zet.T, zft.T)


# speedup vs baseline: 1.1059x; 1.1059x over previous
import jax
import jax.numpy as jnp
import numpy as np
from jax.experimental import pallas as pl
from jax.experimental.pallas import tpu as pltpu

NV, NE, NF = 4096, 8192, 4096
BM = 1024
BK = 2048
NSLOT = 4

_G_SHAPES = [(NV, NV), (NE, NV), (NE, NE), (NF, NE), (NF, NF)]
_X_OFF = [0, 0, NV, NV, NV + NE]


def _build_schedule():
    seg, roff, coff, blk, xrow, firstk = [], [], [], [], [], []
    blk_base = 0
    for g, (m, kdim) in enumerate(_G_SHAPES):
        n_i, n_k = m // BM, kdim // BK
        for i in range(n_i):
            for k in range(n_k):
                seg.append(g)
                roff.append(i * BM)
                coff.append(k * BK)
                blk.append(blk_base + i)
                xrow.append(_X_OFF[g] + k * BK)
                firstk.append(1 if k == 0 else 0)
        blk_base += n_i
    pad = [0] * NSLOT
    arrs = [seg, roff, coff, blk, xrow, firstk]
    return [np.asarray(a + pad, dtype=np.int32) for a in arrs], blk_base


_SCHED, _NBLK = _build_schedule()
_NSTEP = len(_SCHED[0]) - NSLOT


def _big_kernel(seg_ref, roff_ref, coff_ref, blk_ref, xrow_ref, fk_ref,
                xall_ref, g0_ref, g1_ref, g2_ref, g3_ref, g4_ref,
                t_ref, buf_ref, sem_ref):
    s = pl.program_id(0)
    g_refs = [g0_ref, g1_ref, g2_ref, g3_ref, g4_ref]

    def enqueue(t, slot):
        for c in range(5):
            @pl.when(seg_ref[t] == c)
            def _(c=c):
                src = g_refs[c].at[pl.ds(pl.multiple_of(roff_ref[t], BM), BM),
                                   pl.ds(pl.multiple_of(coff_ref[t], BK), BK)]
                pltpu.make_async_copy(src, buf_ref.at[slot],
                                      sem_ref.at[slot]).start()

    @pl.when(s == 0)
    def _():
        for j in range(NSLOT):
            enqueue(j, j)

    slot = jax.lax.rem(s, NSLOT)
    pltpu.make_async_copy(
        g0_ref.at[pl.ds(0, BM), pl.ds(0, BK)],
        buf_ref.at[slot], sem_ref.at[slot]).wait()

    g16 = buf_ref[slot].astype(jnp.bfloat16)
    x_blk = xall_ref[pl.ds(pl.multiple_of(xrow_ref[s], BK), BK), :]
    part = jax.lax.dot_general(
        x_blk, g16,
        dimension_numbers=(((0,), (1,)), ((), ())),
        preferred_element_type=jnp.float32)

    b = blk_ref[s]
    prev = jnp.where(fk_ref[s] == 1, jnp.zeros_like(part), t_ref[b])
    t_ref[b] = prev + part

    @pl.when(s + NSLOT < _NSTEP)
    def _():
        enqueue(s + NSLOT, slot)


def _w_apply_kernel(t_ref, wv_ref, we1_ref, we2_ref, wf1_ref, wf2_ref,
                    ov_ref, oe_ref, of_ref):
    def wt(w_ref, c):
        return jax.lax.dot_general(
            w_ref[...].astype(jnp.bfloat16),
            t_ref[c].astype(jnp.bfloat16),
            dimension_numbers=(((0,), (0,)), ((), ())),
            preferred_element_type=jnp.float32)

    nv_b, ne_b, nf_b = NV // BM, NE // BM, NF // BM
    o = 0
    for j in range(nv_b):
        ov_ref[:, pl.ds(j * BM, BM)] = wt(wv_ref, o + j)
    o += nv_b
    for j in range(ne_b):
        oe_ref[:, pl.ds(j * BM, BM)] = (wt(we1_ref, o + j)
                                        + wt(we2_ref, o + ne_b + j))
    o += 2 * ne_b
    for j in range(nf_b):
        of_ref[:, pl.ds(j * BM, BM)] = (wt(wf1_ref, o + j)
                                        + wt(wf2_ref, o + nf_b + j))


@jax.jit
def kernel(xv, xe, xf, Gv2v, Gv2e, Ge2e, Ge2f, Gf2f, Wv2v, Wve, Wee, Wef, Wff):
    xall = jnp.concatenate([xv, xe, xf], axis=0).astype(jnp.bfloat16)

    hbm_spec = pl.BlockSpec(memory_space=pltpu.MemorySpace.HBM)
    t_all = pl.pallas_call(
        _big_kernel,
        grid_spec=pltpu.PrefetchScalarGridSpec(
            num_scalar_prefetch=6,
            grid=(_NSTEP,),
            in_specs=[
                pl.BlockSpec((NV + NE + NF, 32), lambda s, *_: (0, 0)),
                hbm_spec, hbm_spec, hbm_spec, hbm_spec, hbm_spec,
            ],
            out_specs=pl.BlockSpec((_NBLK, 32, BM), lambda s, *_: (0, 0, 0)),
            scratch_shapes=[
                pltpu.VMEM((NSLOT, BM, BK), jnp.float32),
                pltpu.SemaphoreType.DMA((NSLOT,)),
            ],
        ),
        out_shape=jax.ShapeDtypeStruct((_NBLK, 32, BM), jnp.float32),
        compiler_params=pltpu.CompilerParams(
            dimension_semantics=("arbitrary",),
        ),
    )(*_SCHED, xall, Gv2v, Gv2e, Ge2e, Ge2f, Gf2f)

    zvt, zet, zft = pl.pallas_call(
        _w_apply_kernel,
        out_shape=(
            jax.ShapeDtypeStruct((32, NV), jnp.float32),
            jax.ShapeDtypeStruct((32, NE), jnp.float32),
            jax.ShapeDtypeStruct((32, NF), jnp.float32),
        ),
    )(t_all, Wv2v, Wve, Wee, Wef, Wff)
    return (zvt.T, zet.T, zft.T)


# speedup vs baseline: 1.1065x; 1.0005x over previous
import jax
import jax.numpy as jnp
import numpy as np
from jax.experimental import pallas as pl
from jax.experimental.pallas import tpu as pltpu

NV, NE, NF = 4096, 8192, 4096
BM = 1024
BK = 2048
NSLOT = 4

_G_SHAPES = [(NV, NV), (NE, NV), (NE, NE), (NF, NE), (NF, NF)]
_X_OFF = [0, 0, NV, NV, NV + NE]


def _build_schedule():
    seg, roff, coff, blk, xrow, firstk = [], [], [], [], [], []
    blk_base = 0
    for g, (m, kdim) in enumerate(_G_SHAPES):
        n_i, n_k = m // BM, kdim // BK
        for i in range(n_i):
            for k in range(n_k):
                seg.append(g)
                roff.append(i * BM)
                coff.append(k * BK)
                blk.append(blk_base + i)
                xrow.append(_X_OFF[g] + k * BK)
                firstk.append(1 if k == 0 else 0)
        blk_base += n_i
    pad = [0] * NSLOT
    arrs = [seg, roff, coff, blk, xrow, firstk]
    return [np.asarray(a + pad, dtype=np.int32) for a in arrs], blk_base


_SCHED, _NBLK = _build_schedule()
_NSTEP = len(_SCHED[0]) - NSLOT


def _big_kernel(seg_ref, roff_ref, coff_ref, blk_ref, xrow_ref, fk_ref,
                xall_ref, g0_ref, g1_ref, g2_ref, g3_ref, g4_ref,
                t_ref, buf_ref, sem_ref):
    s = pl.program_id(0)
    g_refs = [g0_ref, g1_ref, g2_ref, g3_ref, g4_ref]

    def enqueue(t, slot):
        half = BM // 2
        for c in range(5):
            @pl.when(seg_ref[t] == c)
            def _(c=c):
                r0 = pl.multiple_of(roff_ref[t], BM)
                c0 = pl.multiple_of(coff_ref[t], BK)
                src_lo = g_refs[c].at[pl.ds(r0, half), pl.ds(c0, BK)]
                src_hi = g_refs[c].at[pl.ds(r0 + half, half), pl.ds(c0, BK)]
                pltpu.make_async_copy(
                    src_lo, buf_ref.at[slot, pl.ds(0, half)],
                    sem_ref.at[slot]).start()
                pltpu.make_async_copy(
                    src_hi, buf_ref.at[slot, pl.ds(half, half)],
                    sem_ref.at[slot]).start()

    @pl.when(s == 0)
    def _():
        for j in range(NSLOT):
            enqueue(j, j)

    slot = jax.lax.rem(s, NSLOT)
    half = BM // 2
    for h in range(2):
        pltpu.make_async_copy(
            g0_ref.at[pl.ds(h * half, half), pl.ds(0, BK)],
            buf_ref.at[slot, pl.ds(h * half, half)],
            sem_ref.at[slot]).wait()

    g16 = buf_ref[slot].astype(jnp.bfloat16)
    x_blk = xall_ref[pl.ds(pl.multiple_of(xrow_ref[s], BK), BK), :]
    part = jax.lax.dot_general(
        x_blk, g16,
        dimension_numbers=(((0,), (1,)), ((), ())),
        preferred_element_type=jnp.float32)

    b = blk_ref[s]
    prev = jnp.where(fk_ref[s] == 1, jnp.zeros_like(part), t_ref[b])
    t_ref[b] = prev + part

    @pl.when(s + NSLOT < _NSTEP)
    def _():
        enqueue(s + NSLOT, slot)


def _w_apply_kernel(t_ref, wv_ref, we1_ref, we2_ref, wf1_ref, wf2_ref,
                    ov_ref, oe_ref, of_ref):
    def wt(w_ref, c):
        return jax.lax.dot_general(
            w_ref[...].astype(jnp.bfloat16),
            t_ref[c].astype(jnp.bfloat16),
            dimension_numbers=(((0,), (0,)), ((), ())),
            preferred_element_type=jnp.float32)

    nv_b, ne_b, nf_b = NV // BM, NE // BM, NF // BM
    o = 0
    for j in range(nv_b):
        ov_ref[:, pl.ds(j * BM, BM)] = wt(wv_ref, o + j)
    o += nv_b
    for j in range(ne_b):
        oe_ref[:, pl.ds(j * BM, BM)] = (wt(we1_ref, o + j)
                                        + wt(we2_ref, o + ne_b + j))
    o += 2 * ne_b
    for j in range(nf_b):
        of_ref[:, pl.ds(j * BM, BM)] = (wt(wf1_ref, o + j)
                                        + wt(wf2_ref, o + nf_b + j))


@jax.jit
def kernel(xv, xe, xf, Gv2v, Gv2e, Ge2e, Ge2f, Gf2f, Wv2v, Wve, Wee, Wef, Wff):
    xall = jnp.concatenate([xv, xe, xf], axis=0).astype(jnp.bfloat16)

    hbm_spec = pl.BlockSpec(memory_space=pltpu.MemorySpace.HBM)
    t_all = pl.pallas_call(
        _big_kernel,
        grid_spec=pltpu.PrefetchScalarGridSpec(
            num_scalar_prefetch=6,
            grid=(_NSTEP,),
            in_specs=[
                pl.BlockSpec((NV + NE + NF, 32), lambda s, *_: (0, 0)),
                hbm_spec, hbm_spec, hbm_spec, hbm_spec, hbm_spec,
            ],
            out_specs=pl.BlockSpec((_NBLK, 32, BM), lambda s, *_: (0, 0, 0)),
            scratch_shapes=[
                pltpu.VMEM((NSLOT, BM, BK), jnp.float32),
                pltpu.SemaphoreType.DMA((NSLOT,)),
            ],
        ),
        out_shape=jax.ShapeDtypeStruct((_NBLK, 32, BM), jnp.float32),
        compiler_params=pltpu.CompilerParams(
            dimension_semantics=("arbitrary",),
        ),
    )(*_SCHED, xall, Gv2v, Gv2e, Ge2e, Ge2f, Gf2f)

    zvt, zet, zft = pl.pallas_call(
        _w_apply_kernel,
        out_shape=(
            jax.ShapeDtypeStruct((32, NV), jnp.float32),
            jax.ShapeDtypeStruct((32, NE), jnp.float32),
            jax.ShapeDtypeStruct((32, NF), jnp.float32),
        ),
    )(t_all, Wv2v, Wve, Wee, Wef, Wff)
    return (zvt.T, zet.T, zft.T)
